# baseline (speedup 1.0000x reference)
import jax
import jax.numpy as jnp
from jax import lax
from jax.experimental import pallas as pl
from jax.experimental.pallas import tpu as pltpu
from jax.experimental.pallas import tpu_sc as plsc

_Q = 8
_V = 1000
_D = 64
_B = 1024
_T = 50

_NC = 2
_NS = 16
_NW = _NC * _NS
_REPS = _NW // _Q
_BPW = _B // _REPS
_LANES = 16
_SBB = 8
_SBL = _SBB * _T
_NGRP = _SBL // _LANES
_NSB = _BPW // _SBB


def _sc_body(seq_hbm, tab_hbm, out_hbm, tab_v, idx0, idx1, stg0, stg1,
             sem_s0, sem_s1, sem_o0, sem_o1):
    idx = (idx0, idx1)
    stg = (stg0, stg1)
    sem_s = (sem_s0, sem_s1)
    sem_o = (sem_o0, sem_o1)

    s = lax.axis_index("s")
    c_ax = lax.axis_index("c")
    q = lax.bitwise_and(s, _Q - 1)
    rep = lax.shift_right_logical(s, 3) * _NC + c_ax
    b0 = rep * _BPW

    tab_base = pl.multiple_of(q * (_V * _D), _V * _D)
    pltpu.sync_copy(tab_hbm.at[pl.ds(tab_base, _V * _D)], tab_v)

    def idx_start(sb):
        return pl.multiple_of(q * (_B * _T) + (b0 + sb * _SBB) * _T, _SBL)

    def out_start(sb, bi):
        return ((b0 + sb * _SBB + bi) * _Q + q) * (_T * _D)

    def idx_copy(sb, p):
        return pltpu.async_copy(seq_hbm.at[pl.ds(idx_start(sb), _SBL)],
                                idx[p], sem_s[p])

    def wb(sb, bi, p):
        return pltpu.async_copy(
            stg[p].at[pl.ds(bi * (_T * _D), _T * _D)],
            out_hbm.at[pl.ds(out_start(sb, bi), _T * _D)], sem_o[p])

    idx_copy(0, 0)
    idx_copy(1, 1)
    for p in (0, 1):
        for bi in range(_SBB):
            wb(p, bi, p)

    iota = lax.broadcasted_iota(jnp.int32, (_LANES,), 0)
    siota = iota * _D

    def fill(sb_p):
        sb, p = sb_p

        def grp(j, carry):
            rv = idx[p][pl.ds(j * _LANES, _LANES)]
            rv64 = rv * _D
            srow = siota + j * (_LANES * _D)
            for col in range(_D):
                vals = plsc.load_gather(tab_v, [rv64 + col])
                plsc.store_scatter(stg[p], [srow + col], vals)
            return carry

        lax.fori_loop(0, _NGRP, grp, 0)

    def body(it, carry):
        for p in (0, 1):
            sb = it * 2 + p
            pltpu.make_async_copy(seq_hbm.at[pl.ds(0, _SBL)], idx[p],
                                  sem_s[p]).wait()
            for bi in range(_SBB):
                pltpu.make_async_copy(
                    stg[p].at[pl.ds(bi * (_T * _D), _T * _D)],
                    out_hbm.at[pl.ds(0, _T * _D)], sem_o[p]).wait()
            fill((sb, p))
            for bi in range(_SBB):
                wb(sb, bi, p)

            @pl.when(it * 2 + p + 2 < _NSB)
            def _():
                idx_copy(sb + 2, p)

        return carry

    lax.fori_loop(0, _NSB // 2, body, 0)
    for p in (0, 1):
        for bi in range(_SBB):
            pltpu.make_async_copy(
                stg[p].at[pl.ds(bi * (_T * _D), _T * _D)],
                out_hbm.at[pl.ds(0, _T * _D)], sem_o[p]).wait()


@jax.jit
def kernel(sequence, tables):
    seq_t = jnp.transpose(sequence, (1, 0, 2)).astype(jnp.int32).reshape(-1)
    tab_flat = tables.reshape(-1)
    out_shape = jax.ShapeDtypeStruct((_B * _Q * _T * _D,), jnp.float32)
    mesh = plsc.VectorSubcoreMesh(core_axis_name="c", subcore_axis_name="s")
    call = pl.kernel(
        _sc_body,
        mesh=mesh,
        out_type=out_shape,
        scratch_types=[
            pltpu.VMEM((_V * _D,), jnp.float32),
            pltpu.VMEM((_SBL,), jnp.int32),
            pltpu.VMEM((_SBL,), jnp.int32),
            pltpu.VMEM((_SBL * _D,), jnp.float32),
            pltpu.VMEM((_SBL * _D,), jnp.float32),
            pltpu.SemaphoreType.DMA,
            pltpu.SemaphoreType.DMA,
            pltpu.SemaphoreType.DMA,
            pltpu.SemaphoreType.DMA,
        ],
        compiler_params=pltpu.CompilerParams(use_tc_tiling_on_sc=False, needs_layout_passes=False),
    )
    out = call(seq_t, tab_flat)
    return out.reshape(_B, _Q, _T, _D)


# speedup vs baseline: 2.5423x; 2.5423x over previous
import jax
import jax.numpy as jnp
from jax import lax
from jax.experimental import pallas as pl
from jax.experimental.pallas import tpu as pltpu
from jax.experimental.pallas import tpu_sc as plsc

_Q = 8
_V = 1000
_D = 64
_DP = _D + 1
_B = 1024
_T = 50

_NC = 2
_NS = 16
_NW = _NC * _NS
_REPS = _NW // _Q
_BPW = _B // _REPS
_LANES = 16
_NSEG = _D // _LANES
_SBB = 8
_SBL = _SBB * _T
_NGRP = _SBL // _LANES
_NSB = _BPW // _SBB


def _sc_body(seq_hbm, tab_hbm, out_hbm, tab_v, idx0, idx1, stg0, stg1,
             sem_s0, sem_s1, sem_o0, sem_o1):
    idx = (idx0, idx1)
    stg = (stg0, stg1)
    sem_s = (sem_s0, sem_s1)
    sem_o = (sem_o0, sem_o1)

    s = lax.axis_index("s")
    c_ax = lax.axis_index("c")
    q = lax.bitwise_and(s, _Q - 1)
    rep = lax.shift_right_logical(s, 3) * _NC + c_ax
    b0 = rep * _BPW

    tab_base = pl.multiple_of(q * (_V * _DP), _V * _DP)
    pltpu.sync_copy(tab_hbm.at[pl.ds(tab_base, _V * _DP)], tab_v)

    def idx_start(sb):
        return pl.multiple_of(q * (_B * _T) + (b0 + sb * _SBB) * _T, _SBL)

    def out_start(sb, bi):
        return ((b0 + sb * _SBB + bi) * _Q + q) * (_T * _D)

    def idx_copy(sb, p):
        return pltpu.async_copy(seq_hbm.at[pl.ds(idx_start(sb), _SBL)],
                                idx[p], sem_s[p])

    def wb(sb, bi, p):
        return pltpu.async_copy(
            stg[p].at[pl.ds(bi * (_T * _D), _T * _D)],
            out_hbm.at[pl.ds(out_start(sb, bi), _T * _D)], sem_o[p])

    idx_copy(0, 0)
    idx_copy(1, 1)
    for p in (0, 1):
        for bi in range(_SBB):
            wb(p, bi, p)

    iota = lax.broadcasted_iota(jnp.int32, (_LANES,), 0)
    coff = [iota + k * _LANES for k in range(_NSEG)]
    lsplat = [jnp.full((_LANES, 1), l, jnp.int32) for l in range(_LANES)]
    dnums = lax.GatherDimensionNumbers(offset_dims=(),
                                       collapsed_slice_dims=(0,),
                                       start_index_map=(0,))

    def fill(p):
        def grp(j, carry):
            rv = idx[p][pl.ds(j * _LANES, _LANES)]
            sbase = j * (_LANES * _D)
            for l in range(_LANES):
                r = lax.gather(rv, lsplat[l], dnums, slice_sizes=(1,),
                               mode=lax.GatherScatterMode.PROMISE_IN_BOUNDS)
                rp = r * _DP
                for k in range(_NSEG):
                    vals = plsc.load_gather(tab_v, [rp + coff[k]])
                    stg[p][pl.ds(sbase + l * _D + k * _LANES, _LANES)] = vals
            return carry

        lax.fori_loop(0, _NGRP, grp, 0)

    def body(it, carry):
        for p in (0, 1):
            sb = it * 2 + p
            pltpu.make_async_copy(seq_hbm.at[pl.ds(0, _SBL)], idx[p],
                                  sem_s[p]).wait()
            for bi in range(_SBB):
                pltpu.make_async_copy(
                    stg[p].at[pl.ds(bi * (_T * _D), _T * _D)],
                    out_hbm.at[pl.ds(0, _T * _D)], sem_o[p]).wait()
            fill(p)
            for bi in range(_SBB):
                wb(sb, bi, p)

            @pl.when(it * 2 + p + 2 < _NSB)
            def _():
                idx_copy(sb + 2, p)

        return carry

    lax.fori_loop(0, _NSB // 2, body, 0)
    for p in (0, 1):
        for bi in range(_SBB):
            pltpu.make_async_copy(
                stg[p].at[pl.ds(bi * (_T * _D), _T * _D)],
                out_hbm.at[pl.ds(0, _T * _D)], sem_o[p]).wait()


@jax.jit
def kernel(sequence, tables):
    seq_t = jnp.transpose(sequence, (1, 0, 2)).astype(jnp.int32).reshape(-1)
    tab_pad = jnp.pad(tables, ((0, 0), (0, 0), (0, _DP - _D))).reshape(-1)
    out_shape = jax.ShapeDtypeStruct((_B * _Q * _T * _D,), jnp.float32)
    mesh = plsc.VectorSubcoreMesh(core_axis_name="c", subcore_axis_name="s")
    call = pl.kernel(
        _sc_body,
        mesh=mesh,
        out_type=out_shape,
        scratch_types=[
            pltpu.VMEM((_V * _DP,), jnp.float32),
            pltpu.VMEM((_SBL,), jnp.int32),
            pltpu.VMEM((_SBL,), jnp.int32),
            pltpu.VMEM((_SBL * _D,), jnp.float32),
            pltpu.VMEM((_SBL * _D,), jnp.float32),
            pltpu.SemaphoreType.DMA,
            pltpu.SemaphoreType.DMA,
            pltpu.SemaphoreType.DMA,
            pltpu.SemaphoreType.DMA,
        ],
        compiler_params=pltpu.CompilerParams(use_tc_tiling_on_sc=False,
                                             needs_layout_passes=False),
    )
    out = call(seq_t, tab_pad)
    return out.reshape(_B, _Q, _T, _D)


# speedup vs baseline: 3.3684x; 1.3250x over previous
import jax
import jax.numpy as jnp
from jax import lax
from jax.experimental import pallas as pl
from jax.experimental.pallas import tpu as pltpu
from jax.experimental.pallas import tpu_sc as plsc

_Q = 8
_V = 1000
_D = 64
_DP = _D + 1
_B = 1024
_T = 50

_NC = 2
_NS = 16
_NW = _NC * _NS
_REPS = _NW // _Q
_BPW = _B // _REPS
_LANES = 16
_NSEG = _D // _LANES
_SBB = 8
_SBL = _SBB * _T
_NGRP = _SBL // _LANES
_NSB = _BPW // _SBB


def _sc_body(seq_hbm, tab_hbm, out_hbm, tab_v, idx0, idx1, stg0, stg1,
             sem_s0, sem_s1, sem_o0, sem_o1):
    idx = (idx0, idx1)
    stg = (stg0, stg1)
    sem_s = (sem_s0, sem_s1)
    sem_o = (sem_o0, sem_o1)

    s = lax.axis_index("s")
    c_ax = lax.axis_index("c")
    q = lax.bitwise_and(s, _Q - 1)
    rep = lax.shift_right_logical(s, 3) * _NC + c_ax
    b0 = rep * _BPW

    tab_base = pl.multiple_of(q * (_V * _DP), _V * _DP)
    pltpu.sync_copy(tab_hbm.at[pl.ds(tab_base, _V * _DP)], tab_v)

    def idx_start(sb):
        return pl.multiple_of(q * (_B * _T) + (b0 + sb * _SBB) * _T, _SBL)

    def out_start(sb, bi):
        return ((b0 + sb * _SBB + bi) * _Q + q) * (_T * _D)

    def idx_copy(sb, p):
        return pltpu.async_copy(seq_hbm.at[pl.ds(idx_start(sb), _SBL)],
                                idx[p], sem_s[p])

    def wb(sb, bi, p):
        return pltpu.async_copy(
            stg[p].at[pl.ds(bi * (_T * _D), _T * _D)],
            out_hbm.at[pl.ds(out_start(sb, bi), _T * _D)], sem_o[p])

    idx_copy(0, 0)
    idx_copy(1, 1)
    for p in (0, 1):
        for bi in range(_SBB):
            wb(p, bi, p)

    iota = lax.broadcasted_iota(jnp.int32, (_LANES,), 0)
    coff = [iota + k * _LANES for k in range(_NSEG)]
    lsplat = [jnp.full((_LANES, 1), l, jnp.int32) for l in range(_LANES)]
    dnums = lax.GatherDimensionNumbers(offset_dims=(),
                                       collapsed_slice_dims=(0,),
                                       start_index_map=(0,))

    def fill(p):
        def grp(j, carry):
            rv = idx[p][pl.ds(j * _LANES, _LANES)]
            sbase = j * (_LANES * _D)
            for l0 in range(0, _LANES, 4):
                vals = []
                for l in range(l0, l0 + 4):
                    r = lax.gather(rv, lsplat[l], dnums, slice_sizes=(1,),
                                   mode=lax.GatherScatterMode.PROMISE_IN_BOUNDS)
                    rp = r * _DP
                    vals.append([plsc.load_gather(tab_v, [rp + coff[k]])
                                 for k in range(_NSEG)])
                for li, l in enumerate(range(l0, l0 + 4)):
                    for k in range(_NSEG):
                        stg[p][pl.ds(sbase + l * _D + k * _LANES,
                                     _LANES)] = vals[li][k]
            return carry

        lax.fori_loop(0, _NGRP, grp, 0)

    def body(it, carry):
        for p in (0, 1):
            sb = it * 2 + p
            pltpu.make_async_copy(seq_hbm.at[pl.ds(0, _SBL)], idx[p],
                                  sem_s[p]).wait()
            for bi in range(_SBB):
                pltpu.make_async_copy(
                    stg[p].at[pl.ds(bi * (_T * _D), _T * _D)],
                    out_hbm.at[pl.ds(0, _T * _D)], sem_o[p]).wait()
            fill(p)
            for bi in range(_SBB):
                wb(sb, bi, p)

            @pl.when(it * 2 + p + 2 < _NSB)
            def _():
                idx_copy(sb + 2, p)

        return carry

    lax.fori_loop(0, _NSB // 2, body, 0)
    for p in (0, 1):
        for bi in range(_SBB):
            pltpu.make_async_copy(
                stg[p].at[pl.ds(bi * (_T * _D), _T * _D)],
                out_hbm.at[pl.ds(0, _T * _D)], sem_o[p]).wait()


@jax.jit
def kernel(sequence, tables):
    seq_t = jnp.transpose(sequence, (1, 0, 2)).astype(jnp.int32).reshape(-1)
    tab_pad = jnp.pad(tables, ((0, 0), (0, 0), (0, _DP - _D))).reshape(-1)
    out_shape = jax.ShapeDtypeStruct((_B * _Q * _T * _D,), jnp.float32)
    mesh = plsc.VectorSubcoreMesh(core_axis_name="c", subcore_axis_name="s")
    call = pl.kernel(
        _sc_body,
        mesh=mesh,
        out_type=out_shape,
        scratch_types=[
            pltpu.VMEM((_V * _DP,), jnp.float32),
            pltpu.VMEM((_SBL,), jnp.int32),
            pltpu.VMEM((_SBL,), jnp.int32),
            pltpu.VMEM((_SBL * _D,), jnp.float32),
            pltpu.VMEM((_SBL * _D,), jnp.float32),
            pltpu.SemaphoreType.DMA,
            pltpu.SemaphoreType.DMA,
            pltpu.SemaphoreType.DMA,
            pltpu.SemaphoreType.DMA,
        ],
        compiler_params=pltpu.CompilerParams(use_tc_tiling_on_sc=False,
                                             needs_layout_passes=False),
    )
    out = call(seq_t, tab_pad)
    return out.reshape(_B, _Q, _T, _D)


# speedup vs baseline: 3.6800x; 1.0925x over previous
import numpy as np

import jax
import jax.numpy as jnp
from jax import lax
from jax.experimental import pallas as pl
from jax.experimental.pallas import tpu as pltpu
from jax.experimental.pallas import tpu_sc as plsc

_Q = 8
_V = 1000
_D = 64
_B = 1024
_T = 50

_TOTAL = _B * _Q * _T
_NC = 2
_NS = 16
_NW = _NC * _NS
_PER_W = _TOTAL // _NW
_CHUNK = 128
_NCHUNK = _PER_W // _CHUNK
_LANES = 16
_PERIOD = _Q * _T
_LUT_LEN = _PERIOD + _CHUNK

_LAG = 3
_NROWS = _LAG + 1
_PD = _LAG + 1
_NIDX = _PD + _LAG

_LUT = np.tile(((np.arange(_PERIOD) // _T) % _Q) * _V, 2)[:_LUT_LEN].astype(
    np.int32)


def _sc_body(seq_hbm, lut_hbm, tab_hbm, out_hbm, tab_sh, lut_v, *rest):
    idx = rest[:_NIDX]
    rows = rest[_NIDX:_NIDX + _NROWS]
    sem_i = rest[_NIDX + _NROWS:2 * _NIDX + _NROWS]
    sem_g = rest[2 * _NIDX + _NROWS:2 * _NIDX + 2 * _NROWS]
    sem_o = rest[2 * _NIDX + 2 * _NROWS:2 * _NIDX + 3 * _NROWS]

    wid = lax.axis_index("s") * _NC + lax.axis_index("c")
    base_w = wid * _PER_W
    sid = lax.axis_index("s")
    rows_per_tile = (_Q * _V) // _NS
    tb = pl.multiple_of(sid * rows_per_tile, rows_per_tile)
    pltpu.sync_copy(tab_hbm.at[pl.ds(tb, rows_per_tile)],
                    tab_sh.at[pl.ds(tb, rows_per_tile)])
    pltpu.sync_copy(lut_hbm, lut_v)
    plsc.subcore_barrier()

    def seq_slice(g):
        return seq_hbm.at[pl.ds(pl.multiple_of(base_w + g * _CHUNK, _CHUNK),
                                _CHUNK)]

    def out_slice(g):
        return out_hbm.at[pl.ds(pl.multiple_of(base_w + g * _CHUNK, _CHUNK),
                                _CHUNK)]

    h_i = [None] * _NCHUNK
    h_g = [None] * _NCHUNK
    h_o = [None] * _NCHUNK
    for g0 in range(_PD):
        h_i[g0] = pltpu.async_copy(seq_slice(g0), idx[g0 % _NIDX],
                                   sem_i[g0 % _NIDX])

    def writeback(g):
        h_g[g].wait()
        h_o[g] = pltpu.async_copy(rows[g % _NROWS], out_slice(g),
                                  sem_o[g % _NROWS])

    for g in range(_NCHUNK):
        i = g % _NIDX
        h_i[g].wait()
        phase = (g * _CHUNK) % _PERIOD
        for j in range(_CHUNK // _LANES):
            sl = pl.ds(j * _LANES, _LANES)
            idx[i][sl] = idx[i][sl] + lut_v[pl.ds(phase + j * _LANES, _LANES)]
        if g - _NROWS >= 0:
            h_o[g - _NROWS].wait()
        h_g[g] = pltpu.async_copy(tab_sh.at[idx[i]], rows[g % _NROWS],
                                  sem_g[g % _NROWS])
        if g - _LAG >= 0:
            writeback(g - _LAG)
        if g + _PD < _NCHUNK:
            h_i[g + _PD] = pltpu.async_copy(seq_slice(g + _PD),
                                            idx[(g + _PD) % _NIDX],
                                            sem_i[(g + _PD) % _NIDX])

    for g in range(_NCHUNK - _LAG, _NCHUNK):
        writeback(g)
    for g in range(_NCHUNK - _NROWS, _NCHUNK):
        h_o[g].wait()


@jax.jit
def kernel(sequence, tables):
    seq_flat = sequence.reshape(-1).astype(jnp.int32)
    tab_flat = tables.reshape(_Q * _V, _D)
    lut = jnp.asarray(_LUT)
    mesh = plsc.VectorSubcoreMesh(core_axis_name="c", subcore_axis_name="s")
    scratch = [
        pltpu.VMEM_SHARED((_Q * _V, _D), jnp.float32),
        pltpu.VMEM((_LUT_LEN,), jnp.int32),
    ]
    scratch += [pltpu.VMEM((_CHUNK,), jnp.int32) for _ in range(_NIDX)]
    scratch += [pltpu.VMEM((_CHUNK, _D), jnp.float32) for _ in range(_NROWS)]
    scratch += [pltpu.SemaphoreType.DMA] * (_NIDX + 2 * _NROWS)
    call = pl.kernel(
        _sc_body,
        mesh=mesh,
        out_type=jax.ShapeDtypeStruct((_TOTAL, _D), jnp.float32),
        scratch_types=scratch,
        compiler_params=pltpu.CompilerParams(use_tc_tiling_on_sc=False),
    )
    out = call(seq_flat, lut, tab_flat)
    return out.reshape(_B, _Q, _T, _D)
